# Initial kernel scaffold; baseline (speedup 1.0000x reference)
import jax
import jax.numpy as jnp
from jax.experimental import pallas as pl


def kernel(x, edge_index, edge_attr, batch, We0, be0, eps0, Wn0, bn0, g0, bt0, We1, be1, eps1, Wn1, bn1, g1, bt1, Wc, bc, Wf, bf):
    raise NotImplementedError("write your pallas kernel here")


# speedup vs baseline: 2.5651x; 2.5651x over previous
import functools

import jax
import jax.numpy as jnp
from jax import lax
from jax.experimental import pallas as pl
from jax.experimental.pallas import tpu as pltpu
from jax.experimental.pallas import tpu_sc as plsc

N = 10000
E = 320000
F = 128
FE = 16
G = 64
C = 128
H = 128
NCLS = 4

SC_CORES = 2
SC_SUBCORES = 16
NW = SC_CORES * SC_SUBCORES
EPW = E // NW
K = 80
NCHUNK = EPW // K
NPAD = 10240
RPS = NPAD // SC_SUBCORES
ZR = 128
LANES = 16
CV = C // LANES


def _mp_sc(x, e, src, dst):
    mesh = plsc.VectorSubcoreMesh(core_axis_name="c", subcore_axis_name="s")

    @functools.partial(
        pl.kernel,
        out_type=jax.ShapeDtypeStruct((SC_CORES, NPAD, C), jnp.float32),
        mesh=mesh,
        scratch_types=[
            pltpu.VMEM((K,), jnp.int32),
            pltpu.VMEM((K,), jnp.int32),
            pltpu.VMEM((K, C), jnp.float32),
            pltpu.VMEM((K, C), jnp.float32),
            pltpu.VMEM((ZR, C), jnp.float32),
            pltpu.VMEM_SHARED((NPAD, C), jnp.float32),
            pltpu.SemaphoreType.DMA,
        ],
    )
    def mp(x_hbm, e_hbm, src_hbm, dst_hbm, out_hbm,
           src_v, dst_v, m_v, e_v, z_v, acc_sh, sem):
        cid = lax.axis_index("c")
        sid = lax.axis_index("s")
        wid = sid * SC_CORES + cid
        base = wid * EPW

        zero = jnp.zeros((LANES,), jnp.float32)

        def zrow(r, carry):
            for cc in range(CV):
                z_v[r, pl.ds(cc * LANES, LANES)] = zero
            return carry

        lax.fori_loop(0, ZR, zrow, 0)
        for t in range(RPS // ZR):
            pltpu.sync_copy(z_v, acc_sh.at[pl.ds(sid * RPS + t * ZR, ZR)])
        plsc.subcore_barrier()

        def chunk(j, carry):
            off = base + j * K
            pltpu.sync_copy(src_hbm.at[pl.ds(off, K)], src_v)
            pltpu.sync_copy(dst_hbm.at[pl.ds(off, K)], dst_v)
            pltpu.async_copy(x_hbm.at[src_v], m_v, sem).wait()
            pltpu.sync_copy(e_hbm.at[pl.ds(off, K)], e_v)

            def row(r, c2):
                for cc in range(CV):
                    sl = pl.ds(cc * LANES, LANES)
                    m_v[r, sl] = jnp.maximum(m_v[r, sl] + e_v[r, sl], 0.0)
                return c2

            lax.fori_loop(0, K, row, 0)
            pltpu.sync_copy(m_v, acc_sh.at[dst_v], add=True)
            return carry

        lax.fori_loop(0, NCHUNK, chunk, 0)
        plsc.subcore_barrier()
        pltpu.sync_copy(acc_sh.at[pl.ds(sid * RPS, RPS)],
                        out_hbm.at[cid, pl.ds(sid * RPS, RPS)])

    return mp(x, e, src, dst)


def _edge_embed(edge_attr, We0, be0, We1, be1):
    BE = 4000

    def body(ea_ref, w0_ref, b0_ref, w1_ref, b1_ref, e0_ref, e1_ref):
        ea = ea_ref[...]
        e0_ref[...] = jnp.dot(ea, w0_ref[...],
                              preferred_element_type=jnp.float32) + b0_ref[...]
        e1_ref[...] = jnp.dot(ea, w1_ref[...],
                              preferred_element_type=jnp.float32) + b1_ref[...]

    return pl.pallas_call(
        body,
        grid=(E // BE,),
        in_specs=[
            pl.BlockSpec((BE, FE), lambda i: (i, 0)),
            pl.BlockSpec((FE, C), lambda i: (0, 0)),
            pl.BlockSpec((1, C), lambda i: (0, 0)),
            pl.BlockSpec((FE, C), lambda i: (0, 0)),
            pl.BlockSpec((1, C), lambda i: (0, 0)),
        ],
        out_specs=[
            pl.BlockSpec((BE, C), lambda i: (i, 0)),
            pl.BlockSpec((BE, C), lambda i: (i, 0)),
        ],
        out_shape=[
            jax.ShapeDtypeStruct((E, C), jnp.float32),
            jax.ShapeDtypeStruct((E, C), jnp.float32),
        ],
    )(edge_attr, We0, be0, We1, be1)


def _post(xin, part, Wn, bn, g, bt, scale):
    BN = 2000

    def body(s_ref, x_ref, p_ref, w_ref, b_ref, g_ref, t_ref, o_ref):
        h = s_ref[0] * x_ref[...] + p_ref[0] + p_ref[1]
        hh = jnp.dot(h, w_ref[...], preferred_element_type=jnp.float32) + b_ref[...]
        mu = jnp.mean(hh, axis=-1, keepdims=True)
        d = hh - mu
        var = jnp.mean(d * d, axis=-1, keepdims=True)
        y = d * lax.rsqrt(var + 1e-5) * g_ref[...] + t_ref[...]
        o_ref[...] = jnp.where(y > 0, y, 0.01 * y)

    return pl.pallas_call(
        body,
        grid=(N // BN,),
        in_specs=[
            pl.BlockSpec(memory_space=pltpu.SMEM),
            pl.BlockSpec((BN, C), lambda i: (i, 0)),
            pl.BlockSpec((SC_CORES, BN, C), lambda i: (0, i, 0)),
            pl.BlockSpec((C, C), lambda i: (0, 0)),
            pl.BlockSpec((1, C), lambda i: (0, 0)),
            pl.BlockSpec((1, C), lambda i: (0, 0)),
            pl.BlockSpec((1, C), lambda i: (0, 0)),
        ],
        out_specs=pl.BlockSpec((BN, C), lambda i: (i, 0)),
        out_shape=jax.ShapeDtypeStruct((N, C), jnp.float32),
    )(scale, xin, part, Wn, bn, g, bt)


def _pool(h2, batchf):
    def body(h_ref, b_ref, o_ref):
        gids = lax.broadcasted_iota(jnp.int32, (N, G), 1).astype(jnp.float32)
        onehot = (b_ref[...] == gids).astype(jnp.float32)
        o_ref[...] = lax.dot_general(
            onehot, h_ref[...], (((0,), (0,)), ((), ())),
            preferred_element_type=jnp.float32)

    return pl.pallas_call(
        body,
        in_specs=[
            pl.BlockSpec((N, C), lambda: (0, 0)),
            pl.BlockSpec((N, 1), lambda: (0, 0)),
        ],
        out_specs=pl.BlockSpec((G, C), lambda: (0, 0)),
        out_shape=jax.ShapeDtypeStruct((G, C), jnp.float32),
    )(h2, batchf)


def _classifier(h1, h2, h_pool, batchf, Wc, bc, Wf, bf):
    BN = 2000

    def body(h1_ref, h2_ref, hp_ref, b_ref, wc_ref, bc_ref, wf_ref, bf_ref, o_ref):
        gids = lax.broadcasted_iota(jnp.int32, (BN, G), 1).astype(jnp.float32)
        onehot = (b_ref[...] == gids).astype(jnp.float32)
        hp = jnp.dot(onehot, hp_ref[...], preferred_element_type=jnp.float32)
        wc = wc_ref[...]
        y = (jnp.dot(h1_ref[...], wc[0:C], preferred_element_type=jnp.float32)
             + jnp.dot(h2_ref[...], wc[C:2 * C], preferred_element_type=jnp.float32)
             + jnp.dot(hp, wc[2 * C:3 * C], preferred_element_type=jnp.float32)
             + bc_ref[...])
        y = jnp.where(y > 0, y, 0.01 * y)
        z = jnp.dot(y, wf_ref[...], preferred_element_type=jnp.float32) + bf_ref[...]
        z = z - jnp.max(z, axis=-1, keepdims=True)
        ez = jnp.exp(z)
        o_ref[...] = ez / jnp.sum(ez, axis=-1, keepdims=True)

    return pl.pallas_call(
        body,
        grid=(N // BN,),
        in_specs=[
            pl.BlockSpec((BN, C), lambda i: (i, 0)),
            pl.BlockSpec((BN, C), lambda i: (i, 0)),
            pl.BlockSpec((G, C), lambda i: (0, 0)),
            pl.BlockSpec((BN, 1), lambda i: (i, 0)),
            pl.BlockSpec((3 * C, H), lambda i: (0, 0)),
            pl.BlockSpec((1, H), lambda i: (0, 0)),
            pl.BlockSpec((H, NCLS), lambda i: (0, 0)),
            pl.BlockSpec((1, NCLS), lambda i: (0, 0)),
        ],
        out_specs=pl.BlockSpec((BN, NCLS), lambda i: (i, 0)),
        out_shape=jax.ShapeDtypeStruct((N, NCLS), jnp.float32),
    )(h1, h2, h_pool, batchf, Wc, bc, Wf, bf)


def kernel(x, edge_index, edge_attr, batch,
           We0, be0, eps0, Wn0, bn0, g0, bt0,
           We1, be1, eps1, Wn1, bn1, g1, bt1,
           Wc, bc, Wf, bf):
    src = edge_index[0]
    dst = edge_index[1]
    batchf = batch.astype(jnp.float32).reshape(N, 1)
    r = lambda v: v.reshape(1, -1)

    e0, e1 = _edge_embed(edge_attr, We0, r(be0), We1, r(be1))

    part0 = _mp_sc(x, e0, src, dst)
    h1 = _post(x, part0, Wn0, r(bn0), r(g0), r(bt0),
               (1.0 + eps0).reshape(1))

    part1 = _mp_sc(h1, e1, src, dst)
    h2 = _post(h1, part1, Wn1, r(bn1), r(g1), r(bt1),
               (1.0 + eps1).reshape(1))

    h_pool = _pool(h2, batchf)
    return _classifier(h1, h2, h_pool, batchf, Wc, r(bc), Wf, r(bf))
